# baseline (speedup 1.0000x reference)
import functools
import jax
import jax.numpy as jnp
from jax import lax
from jax.experimental import pallas as pl
from jax.experimental.pallas import tpu as pltpu
from jax.experimental.pallas import tpu_sc as plsc

VOCAB = 1000000
EMBED_DIM = 64
NUM_CLASSES = 128
PAD_IDX = 0
BATCH = 4096
SEQ_LEN = 200

NUM_WORKERS = 32
BPW = BATCH // NUM_WORKERS
SCHUNK = 100
NCHUNK = SEQ_LEN // SCHUNK
NVREG = EMBED_DIM // 16


def _sc_body(text_hbm, table_hbm, out_hbm, idx_v, rows_v, sums_v, sem):
    cid = lax.axis_index("c")
    sid = lax.axis_index("s")
    wid = sid * 2 + cid
    base = wid * BPW
    pltpu.sync_copy(text_hbm.at[pl.ds(base, BPW)], idx_v)

    def row_body(r, _):
        for ch in range(NCHUNK):
            pltpu.async_copy(
                table_hbm.at[idx_v.at[r, ch]],
                rows_v.at[pl.ds(ch * SCHUNK, SCHUNK)],
                sem,
            ).wait()

        def seq_body(s, accs):
            return tuple(
                accs[j] + rows_v[s, pl.ds(j * 16, 16)] for j in range(NVREG)
            )

        zeros = tuple(jnp.zeros((16,), jnp.float32) for _ in range(NVREG))
        accs = lax.fori_loop(0, SEQ_LEN, seq_body, zeros)
        for j in range(NVREG):
            sums_v[r, pl.ds(j * 16, 16)] = accs[j]
        return 0

    lax.fori_loop(0, BPW, row_body, 0)
    pltpu.sync_copy(sums_v, out_hbm.at[pl.ds(base, BPW)])


@functools.partial(jax.jit, static_argnums=())
def _sc_sums(text3, emb_table):
    mesh = plsc.VectorSubcoreMesh(core_axis_name="c", subcore_axis_name="s")
    return pl.kernel(
        _sc_body,
        mesh=mesh,
        out_type=jax.ShapeDtypeStruct((BATCH, EMBED_DIM), jnp.float32),
        scratch_types=[
            pltpu.VMEM((BPW, NCHUNK, SCHUNK), jnp.int32),
            pltpu.VMEM((SEQ_LEN, EMBED_DIM), jnp.float32),
            pltpu.VMEM((BPW, EMBED_DIM), jnp.float32),
            pltpu.SemaphoreType.DMA,
        ],
        compiler_params=pltpu.CompilerParams(use_tc_tiling_on_sc=False),
    )(text3, emb_table)


BB = 256


def _tc_finish_body(text_ref, sums_ref, w_ref, b_ref, out_ref):
    mask = (text_ref[...] != PAD_IDX).astype(jnp.float32)
    cnt = jnp.sum(mask, axis=1, keepdims=True)
    avg = sums_ref[...] / (cnt + 1e-6)
    out_ref[...] = (
        lax.dot_general(
            avg, w_ref[...], (((1,), (1,)), ((), ())),
            preferred_element_type=jnp.float32,
        )
        + b_ref[...]
    )


def _tc_finish(text, sums, fc_w, fc_b2):
    return pl.pallas_call(
        _tc_finish_body,
        grid=(BATCH // BB,),
        in_specs=[
            pl.BlockSpec((BB, SEQ_LEN), lambda i: (i, 0)),
            pl.BlockSpec((BB, EMBED_DIM), lambda i: (i, 0)),
            pl.BlockSpec((NUM_CLASSES, EMBED_DIM), lambda i: (0, 0)),
            pl.BlockSpec((1, NUM_CLASSES), lambda i: (0, 0)),
        ],
        out_specs=pl.BlockSpec((BB, NUM_CLASSES), lambda i: (i, 0)),
        out_shape=jax.ShapeDtypeStruct((BATCH, NUM_CLASSES), jnp.float32),
    )(text, sums, fc_w, fc_b2)


def kernel(text, emb_table, fc_w, fc_b):
    text3 = text.reshape(BATCH, NCHUNK, SCHUNK)
    sums = _sc_sums(text3, emb_table)
    return _tc_finish(text, sums, fc_w, fc_b.reshape(1, NUM_CLASSES))


# speedup vs baseline: 1.2535x; 1.2535x over previous
import functools
import jax
import jax.numpy as jnp
from jax import lax
from jax.experimental import pallas as pl
from jax.experimental.pallas import tpu as pltpu
from jax.experimental.pallas import tpu_sc as plsc

VOCAB = 1000000
EMBED_DIM = 64
NUM_CLASSES = 128
PAD_IDX = 0
BATCH = 4096
SEQ_LEN = 200

NUM_WORKERS = 32
BPW = BATCH // NUM_WORKERS
SCHUNK = 100
NCHUNK = SEQ_LEN // SCHUNK
NVREG = EMBED_DIM // 16


def _sc_body(text_hbm, table_hbm, out_hbm, idx_v, rows_v, sums_v, sems):
    cid = lax.axis_index("c")
    sid = lax.axis_index("s")
    wid = sid * 2 + cid
    base = wid * BPW
    pltpu.sync_copy(text_hbm.at[pl.ds(base, BPW)], idx_v)

    def fire(r, buf):
        for ch in range(NCHUNK):
            pltpu.async_copy(
                table_hbm.at[idx_v.at[r, ch]],
                rows_v.at[buf, pl.ds(ch * SCHUNK, SCHUNK)],
                sems.at[buf],
            )

    fire(0, 0)

    def row_body(r, _):
        buf = lax.rem(r, 2)

        @pl.when(r + 1 < BPW)
        def _prefetch():
            fire(r + 1, 1 - buf)

        pltpu.make_async_copy(
            table_hbm.at[pl.ds(0, SEQ_LEN)], rows_v.at[buf], sems.at[buf]
        ).wait()

        def seq_body(s, accs):
            return tuple(
                accs[j] + rows_v[buf, s, pl.ds(j * 16, 16)]
                for j in range(NVREG)
            )

        zeros = tuple(jnp.zeros((16,), jnp.float32) for _ in range(NVREG))
        accs = plsc.parallel_loop(0, SEQ_LEN, carry=zeros, unroll=8)(seq_body)
        for j in range(NVREG):
            sums_v[r, pl.ds(j * 16, 16)] = accs[j]
        return 0

    lax.fori_loop(0, BPW, row_body, 0)
    pltpu.sync_copy(sums_v, out_hbm.at[pl.ds(base, BPW)])


@functools.partial(jax.jit, static_argnums=())
def _sc_sums(text3, emb_table):
    mesh = plsc.VectorSubcoreMesh(core_axis_name="c", subcore_axis_name="s")
    return pl.kernel(
        _sc_body,
        mesh=mesh,
        out_type=jax.ShapeDtypeStruct((BATCH, EMBED_DIM), jnp.float32),
        scratch_types=[
            pltpu.VMEM((BPW, NCHUNK, SCHUNK), jnp.int32),
            pltpu.VMEM((2, SEQ_LEN, EMBED_DIM), jnp.float32),
            pltpu.VMEM((BPW, EMBED_DIM), jnp.float32),
            pltpu.SemaphoreType.DMA((2,)),
        ],
        compiler_params=pltpu.CompilerParams(use_tc_tiling_on_sc=False),
    )(text3, emb_table)


BB = 256


def _tc_finish_body(text_ref, sums_ref, w_ref, b_ref, out_ref):
    mask = (text_ref[...] != PAD_IDX).astype(jnp.float32)
    cnt = jnp.sum(mask, axis=1, keepdims=True)
    avg = sums_ref[...] / (cnt + 1e-6)
    out_ref[...] = (
        lax.dot_general(
            avg, w_ref[...], (((1,), (1,)), ((), ())),
            preferred_element_type=jnp.float32,
        )
        + b_ref[...]
    )


def _tc_finish(text, sums, fc_w, fc_b2):
    return pl.pallas_call(
        _tc_finish_body,
        grid=(BATCH // BB,),
        in_specs=[
            pl.BlockSpec((BB, SEQ_LEN), lambda i: (i, 0)),
            pl.BlockSpec((BB, EMBED_DIM), lambda i: (i, 0)),
            pl.BlockSpec((NUM_CLASSES, EMBED_DIM), lambda i: (0, 0)),
            pl.BlockSpec((1, NUM_CLASSES), lambda i: (0, 0)),
        ],
        out_specs=pl.BlockSpec((BB, NUM_CLASSES), lambda i: (i, 0)),
        out_shape=jax.ShapeDtypeStruct((BATCH, NUM_CLASSES), jnp.float32),
    )(text, sums, fc_w, fc_b2)


def kernel(text, emb_table, fc_w, fc_b):
    text3 = text.reshape(BATCH, NCHUNK, SCHUNK)
    sums = _sc_sums(text3, emb_table)
    return _tc_finish(text, sums, fc_w, fc_b.reshape(1, NUM_CLASSES))


# speedup vs baseline: 1.2555x; 1.0016x over previous
import functools
import jax
import jax.numpy as jnp
from jax import lax
from jax.experimental import pallas as pl
from jax.experimental.pallas import tpu as pltpu
from jax.experimental.pallas import tpu_sc as plsc

VOCAB = 1000000
EMBED_DIM = 64
NUM_CLASSES = 128
PAD_IDX = 0
BATCH = 4096
SEQ_LEN = 200

NUM_WORKERS = 32
BPW = BATCH // NUM_WORKERS
SCHUNK = 100
NCHUNK = SEQ_LEN // SCHUNK
NVREG = EMBED_DIM // 16


def _sc_body(text_hbm, table_hbm, out_hbm, idx_v, rows_v, sums_v, sems):
    cid = lax.axis_index("c")
    sid = lax.axis_index("s")
    wid = sid * 2 + cid
    base = wid * BPW
    pltpu.sync_copy(text_hbm.at[pl.ds(base, BPW)], idx_v)

    def fire(r, buf):
        for ch in range(NCHUNK):
            pltpu.async_copy(
                table_hbm.at[idx_v.at[r, ch]],
                rows_v.at[buf, pl.ds(ch * SCHUNK, SCHUNK)],
                sems.at[buf],
            )

    fire(0, 0)

    def row_body(r, _):
        buf = lax.rem(r, 2)

        @pl.when(r + 1 < BPW)
        def _prefetch():
            fire(r + 1, 1 - buf)

        pltpu.make_async_copy(
            table_hbm.at[pl.ds(0, SEQ_LEN)], rows_v.at[buf], sems.at[buf]
        ).wait()

        def seq_body(s, accs):
            return tuple(
                accs[j] + rows_v[buf, s, pl.ds(j * 16, 16)]
                for j in range(NVREG)
            )

        zeros = tuple(jnp.zeros((16,), jnp.float32) for _ in range(NVREG))
        accs = plsc.parallel_loop(0, SEQ_LEN, carry=zeros, unroll=8)(seq_body)
        for j in range(NVREG):
            sums_v[r, pl.ds(j * 16, 16)] = accs[j]
        return 0

    lax.fori_loop(0, BPW, row_body, 0)
    pltpu.sync_copy(sums_v, out_hbm.at[pl.ds(base, BPW), pl.ds(0, EMBED_DIM)])


@functools.partial(jax.jit, static_argnums=())
def _sc_sums(text3, emb_table):
    mesh = plsc.VectorSubcoreMesh(core_axis_name="c", subcore_axis_name="s")
    return pl.kernel(
        _sc_body,
        mesh=mesh,
        out_type=jax.ShapeDtypeStruct((BATCH, 2 * EMBED_DIM), jnp.float32),
        scratch_types=[
            pltpu.VMEM((BPW, NCHUNK, SCHUNK), jnp.int32),
            pltpu.VMEM((2, SEQ_LEN, EMBED_DIM), jnp.float32),
            pltpu.VMEM((BPW, EMBED_DIM), jnp.float32),
            pltpu.SemaphoreType.DMA((2,)),
        ],
        compiler_params=pltpu.CompilerParams(use_tc_tiling_on_sc=False),
    )(text3, emb_table)


BB = 256


def _tc_finish_body(text_ref, sums_ref, w_ref, b_ref, out_ref):
    mask = (text_ref[...] != PAD_IDX).astype(jnp.float32)
    cnt = jnp.sum(mask, axis=1, keepdims=True)
    avg = sums_ref[...][:, :EMBED_DIM] / (cnt + 1e-6)
    out_ref[...] = (
        lax.dot_general(
            avg, w_ref[...], (((1,), (1,)), ((), ())),
            preferred_element_type=jnp.float32,
        )
        + b_ref[...]
    )


def _tc_finish(text, sums, fc_w, fc_b2):
    return pl.pallas_call(
        _tc_finish_body,
        grid=(BATCH // BB,),
        in_specs=[
            pl.BlockSpec((BB, SEQ_LEN), lambda i: (i, 0)),
            pl.BlockSpec((BB, 2 * EMBED_DIM), lambda i: (i, 0)),
            pl.BlockSpec((NUM_CLASSES, EMBED_DIM), lambda i: (0, 0)),
            pl.BlockSpec((1, NUM_CLASSES), lambda i: (0, 0)),
        ],
        out_specs=pl.BlockSpec((BB, NUM_CLASSES), lambda i: (i, 0)),
        out_shape=jax.ShapeDtypeStruct((BATCH, NUM_CLASSES), jnp.float32),
    )(text, sums, fc_w, fc_b2)


def kernel(text, emb_table, fc_w, fc_b):
    text3 = text.reshape(BATCH, NCHUNK, SCHUNK)
    sums = _sc_sums(text3, emb_table)
    return _tc_finish(text, sums, fc_w, fc_b.reshape(1, NUM_CLASSES))


# speedup vs baseline: 1.2624x; 1.0055x over previous
import functools
import jax
import jax.numpy as jnp
import numpy as np
from jax import lax
from jax.experimental import pallas as pl
from jax.experimental.pallas import tpu as pltpu
from jax.experimental.pallas import tpu_sc as plsc

VOCAB = 1000000
EMBED_DIM = 64
NUM_CLASSES = 128
PAD_IDX = 0
BATCH = 4096
SEQ_LEN = 200

NUM_WORKERS = 32
BPW = BATCH // NUM_WORKERS
SCHUNKS = ((0, 104), (104, 96))
NVREG = EMBED_DIM // 16


def _sc_body(text_hbm, table_hbm, out_hbm, idx_v, rows_v, sums_v, sems):
    cid = lax.axis_index("c")
    sid = lax.axis_index("s")
    wid = sid * 2 + cid
    base = wid * BPW
    pltpu.sync_copy(text_hbm.at[pl.ds(base, BPW)], idx_v)

    def fire(r, buf):
        for off, size in SCHUNKS:
            pltpu.async_copy(
                table_hbm.at[idx_v.at[r, pl.ds(off, size)]],
                rows_v.at[buf, pl.ds(off, size)],
                sems.at[buf],
            )

    fire(0, 0)

    def row_body(r, _):
        buf = lax.rem(r, 2)

        @pl.when(r + 1 < BPW)
        def _prefetch():
            fire(r + 1, 1 - buf)

        pltpu.make_async_copy(
            table_hbm.at[pl.ds(0, SEQ_LEN)], rows_v.at[buf], sems.at[buf]
        ).wait()

        def seq_body(s, accs):
            return tuple(
                accs[j] + rows_v[buf, s, pl.ds(j * 16, 16)]
                for j in range(NVREG)
            )

        zeros = tuple(jnp.zeros((16,), jnp.float32) for _ in range(NVREG))
        accs = plsc.parallel_loop(0, SEQ_LEN, carry=zeros, unroll=8)(seq_body)
        for j in range(NVREG):
            sums_v[r, pl.ds(j * 16, 16)] = accs[j]
        return 0

    lax.fori_loop(0, BPW, row_body, 0)
    pltpu.sync_copy(sums_v, out_hbm.at[pl.ds(base, BPW), pl.ds(0, EMBED_DIM)])


@functools.partial(jax.jit, static_argnums=())
def _sc_sums(text3, emb_table):
    mesh = plsc.VectorSubcoreMesh(core_axis_name="c", subcore_axis_name="s")
    return pl.kernel(
        _sc_body,
        mesh=mesh,
        out_type=jax.ShapeDtypeStruct((BATCH, 2 * EMBED_DIM), jnp.float32),
        scratch_types=[
            pltpu.VMEM((BPW, SEQ_LEN), jnp.int32),
            pltpu.VMEM((2, SEQ_LEN, EMBED_DIM), jnp.float32),
            pltpu.VMEM((BPW, EMBED_DIM), jnp.float32),
            pltpu.SemaphoreType.DMA((2,)),
        ],
        compiler_params=pltpu.CompilerParams(use_tc_tiling_on_sc=False),
    )(text3, emb_table)


BB = 256


def _tc_finish_body(text_ref, sums_ref, w_ref, b_ref, out_ref):
    mask = (text_ref[...] != PAD_IDX).astype(jnp.float32)
    cnt = jnp.sum(mask, axis=1, keepdims=True)
    avg = sums_ref[...][:, :EMBED_DIM] / (cnt + 1e-6)
    out_ref[...] = (
        lax.dot_general(
            avg, w_ref[...], (((1,), (1,)), ((), ())),
            preferred_element_type=jnp.float32,
        )
        + b_ref[...]
    )


def _tc_finish(text, sums, fc_w, fc_b2):
    return pl.pallas_call(
        _tc_finish_body,
        grid=(BATCH // BB,),
        in_specs=[
            pl.BlockSpec((BB, SEQ_LEN), lambda i: (i, 0)),
            pl.BlockSpec((BB, 2 * EMBED_DIM), lambda i: (i, 0)),
            pl.BlockSpec((NUM_CLASSES, EMBED_DIM), lambda i: (0, 0)),
            pl.BlockSpec((1, NUM_CLASSES), lambda i: (0, 0)),
        ],
        out_specs=pl.BlockSpec((BB, NUM_CLASSES), lambda i: (i, 0)),
        out_shape=jax.ShapeDtypeStruct((BATCH, NUM_CLASSES), jnp.float32),
    )(text, sums, fc_w, fc_b2)


def kernel(text, emb_table, fc_w, fc_b):
    sums = _sc_sums(text, emb_table)
    return _tc_finish(text, sums, fc_w, fc_b.reshape(1, NUM_CLASSES))


# speedup vs baseline: 1.2666x; 1.0033x over previous
import functools
import jax
import jax.numpy as jnp
import numpy as np
from jax import lax
from jax.experimental import pallas as pl
from jax.experimental.pallas import tpu as pltpu
from jax.experimental.pallas import tpu_sc as plsc

VOCAB = 1000000
EMBED_DIM = 64
NUM_CLASSES = 128
PAD_IDX = 0
BATCH = 4096
SEQ_LEN = 200

NUM_WORKERS = 32
BPW = BATCH // NUM_WORKERS
SCHUNKS = ((0, 104), (104, 96))
NVREG = EMBED_DIM // 16


def _sc_body(text_hbm, table_hbm, out_hbm, idx_v, rows_v, sums_v, sems):
    cid = lax.axis_index("c")
    sid = lax.axis_index("s")
    wid = sid * 2 + cid
    base = wid * BPW
    pltpu.sync_copy(text_hbm.at[pl.ds(base * SEQ_LEN, BPW * SEQ_LEN)], idx_v)

    def fire(r, buf):
        for off, size in SCHUNKS:
            pltpu.async_copy(
                table_hbm.at[idx_v.at[pl.ds(r * SEQ_LEN + off, size)]],
                rows_v.at[buf, pl.ds(off, size)],
                sems.at[buf],
            )

    fire(0, 0)

    def row_body(r, _):
        buf = lax.rem(r, 2)

        @pl.when(r + 1 < BPW)
        def _prefetch():
            fire(r + 1, 1 - buf)

        pltpu.make_async_copy(
            table_hbm.at[pl.ds(0, SEQ_LEN)], rows_v.at[buf], sems.at[buf]
        ).wait()

        def seq_body(s, accs):
            return tuple(
                accs[j] + rows_v[buf, s, pl.ds(j * 16, 16)]
                for j in range(NVREG)
            )

        zeros = tuple(jnp.zeros((16,), jnp.float32) for _ in range(NVREG))
        accs = plsc.parallel_loop(0, SEQ_LEN, carry=zeros, unroll=8)(seq_body)
        for j in range(NVREG):
            sums_v[r, pl.ds(j * 16, 16)] = accs[j]
        return 0

    lax.fori_loop(0, BPW, row_body, 0)
    pltpu.sync_copy(sums_v, out_hbm.at[pl.ds(base, BPW), pl.ds(0, EMBED_DIM)])


@functools.partial(jax.jit, static_argnums=())
def _sc_sums(text3, emb_table):
    mesh = plsc.VectorSubcoreMesh(core_axis_name="c", subcore_axis_name="s")
    return pl.kernel(
        _sc_body,
        mesh=mesh,
        out_type=jax.ShapeDtypeStruct((BATCH, 2 * EMBED_DIM), jnp.float32),
        scratch_types=[
            pltpu.VMEM((BPW * SEQ_LEN,), jnp.int32),
            pltpu.VMEM((2, SEQ_LEN, EMBED_DIM), jnp.float32),
            pltpu.VMEM((BPW, EMBED_DIM), jnp.float32),
            pltpu.SemaphoreType.DMA((2,)),
        ],
        compiler_params=pltpu.CompilerParams(use_tc_tiling_on_sc=False),
    )(text3, emb_table)


BB = 256


def _tc_finish_body(text_ref, sums_ref, w_ref, b_ref, out_ref):
    mask = (text_ref[...] != PAD_IDX).astype(jnp.float32)
    cnt = jnp.sum(mask, axis=1, keepdims=True)
    avg = sums_ref[...][:, :EMBED_DIM] / (cnt + 1e-6)
    out_ref[...] = (
        lax.dot_general(
            avg, w_ref[...], (((1,), (1,)), ((), ())),
            preferred_element_type=jnp.float32,
        )
        + b_ref[...]
    )


def _tc_finish(text, sums, fc_w, fc_b2):
    return pl.pallas_call(
        _tc_finish_body,
        grid=(BATCH // BB,),
        in_specs=[
            pl.BlockSpec((BB, SEQ_LEN), lambda i: (i, 0)),
            pl.BlockSpec((BB, 2 * EMBED_DIM), lambda i: (i, 0)),
            pl.BlockSpec((NUM_CLASSES, EMBED_DIM), lambda i: (0, 0)),
            pl.BlockSpec((1, NUM_CLASSES), lambda i: (0, 0)),
        ],
        out_specs=pl.BlockSpec((BB, NUM_CLASSES), lambda i: (i, 0)),
        out_shape=jax.ShapeDtypeStruct((BATCH, NUM_CLASSES), jnp.float32),
    )(text, sums, fc_w, fc_b2)


def kernel(text, emb_table, fc_w, fc_b):
    sums = _sc_sums(text.reshape(BATCH * SEQ_LEN), emb_table)
    return _tc_finish(text, sums, fc_w, fc_b.reshape(1, NUM_CLASSES))
